# baseline (speedup 1.0000x reference)
import jax
import jax.numpy as jnp
from jax.experimental import pallas as pl
from jax.experimental.pallas import tpu as pltpu


def _layer(h, w, b_row, c, relu):
    g = jnp.dot(h, w, preferred_element_type=jnp.float32)
    s = jnp.sum(g, axis=0, keepdims=True)
    g = g * c + (s * c + b_row)
    return jnp.maximum(g, 0.0) if relu else g


def _deep_gcn_body(x_ref, w1_ref, b1_ref, b2_ref, b3_ref, b4_ref,
                   w2_hbm, w3_hbm, w4_hbm, out_ref,
                   w2_v, w3_v, w4_v, sem2, sem3, sem4):
    n = x_ref.shape[0]
    dinv = jax.lax.rsqrt(jnp.float32(n + 1))
    c = dinv * dinv

    cp2 = pltpu.make_async_copy(w2_hbm, w2_v, sem2)
    cp3 = pltpu.make_async_copy(w3_hbm, w3_v, sem3)
    cp4 = pltpu.make_async_copy(w4_hbm, w4_v, sem4)
    cp2.start()
    cp3.start()
    cp4.start()

    h = _layer(x_ref[...], w1_ref[...], b1_ref[...], c, True)
    cp2.wait()
    h = _layer(h, w2_v[...], b2_ref[...], c, True)
    cp3.wait()
    h = _layer(h, w3_v[...], b3_ref[...], c, True)
    cp4.wait()
    out_ref[...] = _layer(h, w4_v[...], b4_ref[...], c, False)


def kernel(x, W1, b1, W2, b2, W3, b3, W4, b4):
    n, _ = x.shape
    d_hid = W2.shape[0]
    d_out = W4.shape[1]
    vmem = pl.BlockSpec(memory_space=pltpu.MemorySpace.VMEM)
    hbm = pl.BlockSpec(memory_space=pl.ANY)
    out = pl.pallas_call(
        _deep_gcn_body,
        out_shape=jax.ShapeDtypeStruct((n, d_out), jnp.float32),
        in_specs=[vmem] * 6 + [hbm] * 3,
        out_specs=vmem,
        scratch_shapes=[
            pltpu.VMEM((d_hid, d_hid), jnp.float32),
            pltpu.VMEM((d_hid, d_hid), jnp.float32),
            pltpu.VMEM((d_hid, d_out), jnp.float32),
            pltpu.SemaphoreType.DMA,
            pltpu.SemaphoreType.DMA,
            pltpu.SemaphoreType.DMA,
        ],
    )(x, W1, b1.reshape(1, -1), b2.reshape(1, -1), b3.reshape(1, -1),
      b4.reshape(1, -1), W2, W3, W4)
    return jnp.squeeze(out)


# speedup vs baseline: 1.1820x; 1.1820x over previous
import jax
import jax.numpy as jnp
from jax.experimental import pallas as pl
from jax.experimental.pallas import tpu as pltpu


def _deep_gcn_body(x_ref, w1_ref, b1_ref, w2_ref, b2_ref, w3_ref, b3_ref,
                   w4_ref, b4_ref, out_ref):
    n = x_ref.shape[0]
    dinv = jax.lax.rsqrt(jnp.float32(n + 1))
    c = dinv * dinv

    cinv = jnp.float32(1.0) / c
    h = x_ref[...]
    bscale = cinv
    for w_ref, b_ref in ((w1_ref, b1_ref), (w2_ref, b2_ref),
                         (w3_ref, b3_ref)):
        g = jnp.dot(h, w_ref[...], preferred_element_type=jnp.float32)
        s = jnp.sum(g, axis=0, keepdims=True)
        h = jnp.maximum(g + (s + b_ref[...] * bscale), 0.0)
        bscale = bscale * cinv
    g = jnp.dot(h, w4_ref[...], preferred_element_type=jnp.float32)
    s = jnp.sum(g, axis=0, keepdims=True)
    c4 = (c * c) * (c * c)
    out_ref[...] = g * c4 + (s * c4 + b4_ref[...])


def kernel(x, W1, b1, W2, b2, W3, b3, W4, b4):
    n, _ = x.shape
    d_out = W4.shape[1]
    out = pl.pallas_call(
        _deep_gcn_body,
        out_shape=jax.ShapeDtypeStruct((n, d_out), jnp.float32),
    )(x, W1, b1.reshape(1, -1), W2, b2.reshape(1, -1),
      W3, b3.reshape(1, -1), W4, b4.reshape(1, -1))
    return jnp.squeeze(out)
